# Initial kernel scaffold; baseline (speedup 1.0000x reference)
import jax
import jax.numpy as jnp
from jax.experimental import pallas as pl


def kernel(species, edge_src, edge_dst, distances, vec, charges, polarisability):
    raise NotImplementedError("write your pallas kernel here")


# speedup vs baseline: 20.6449x; 20.6449x over previous
import functools

import jax
import jax.numpy as jnp
from jax import lax
from jax.experimental import pallas as pl
from jax.experimental.pallas import tpu as pltpu
from jax.experimental.pallas import tpu_sc as plsc

N = 100000
E = 6400000
DAMP = 0.7

NC = 2
NS = 16
NW = NC * NS
EPW = E // NW
C = 2000
NCHUNK = EPW // C

N_PAD = 102400
F_PAD = 300032


def _prep_body(pol_ref, out_ref):
    x = pol_ref[...]
    out_ref[...] = lax.rsqrt(lax.sqrt(x))


def _combine_body(p_ref, out_ref):
    out_ref[...] = p_ref[0] + p_ref[1]


def _nrsqrt(x):
    i = lax.bitcast_convert_type(x, jnp.int32)
    i = jnp.int32(0x5F3759DF) - lax.shift_right_arithmetic(i, 1)
    y = lax.bitcast_convert_type(i, jnp.float32)
    for _ in range(3):
        y = y * (1.5 - 0.5 * x * y * y)
    return y


def _sc_field(src, dst, dist, vecf, charges, pim, zeros):
    mesh = plsc.VectorSubcoreMesh(core_axis_name="c", subcore_axis_name="s")

    @functools.partial(
        pl.kernel,
        mesh=mesh,
        compiler_params=pltpu.CompilerParams(needs_layout_passes=False),
        out_type=jax.ShapeDtypeStruct((NC, 3 * N), jnp.float32),
        scratch_types=[
            pltpu.VMEM((C,), jnp.int32),
            pltpu.VMEM((C,), jnp.int32),
            pltpu.VMEM((C,), jnp.float32),
            pltpu.VMEM((3 * C,), jnp.float32),
            pltpu.VMEM((C,), jnp.float32),
            pltpu.VMEM((C,), jnp.float32),
            pltpu.VMEM((C,), jnp.float32),
            pltpu.VMEM((C,), jnp.float32),
            pltpu.VMEM((3 * C,), jnp.int32),
            pltpu.VMEM((3 * C,), jnp.float32),
            pltpu.VMEM_SHARED((N,), jnp.float32),
            pltpu.VMEM_SHARED((N,), jnp.float32),
            pltpu.VMEM_SHARED((3 * N,), jnp.float32),
            pltpu.SemaphoreType.DMA,
        ],
    )
    def k(src_hbm, dst_hbm, dist_hbm, vec_hbm, charges_hbm, pim_hbm, zeros_hbm,
          out_hbm, src_v, dst_v, dist_v, vec_v, qd, ws, wd, coef, idx3, val3,
          charges_sp, pim_sp, field_sp, sem):
        cid = lax.axis_index("c")
        sid = lax.axis_index("s")
        wid = sid * NC + cid

        @pl.when(sid == 0)
        def _():
            pltpu.sync_copy(charges_hbm, charges_sp)
            pltpu.sync_copy(pim_hbm, pim_sp)
            pltpu.sync_copy(zeros_hbm, field_sp)

        plsc.subcore_barrier()

        ii = lax.iota(jnp.int32, 16)

        def chunk_body(i, carry):
            base = wid * EPW + i * C
            pltpu.sync_copy(src_hbm.at[pl.ds(base, C)], src_v)
            pltpu.sync_copy(dst_hbm.at[pl.ds(base, C)], dst_v)
            pltpu.sync_copy(dist_hbm.at[pl.ds(base, C)], dist_v)
            pltpu.sync_copy(vec_hbm.at[pl.ds(3 * base, 3 * C)], vec_v)
            pltpu.async_copy(charges_sp.at[dst_v], qd, sem).wait()
            pltpu.async_copy(pim_sp.at[src_v], ws, sem).wait()
            pltpu.async_copy(pim_sp.at[dst_v], wd, sem).wait()

            def cbody(j, c2):
                sl = pl.ds(j * 16, 16)
                r = dist_v[sl]
                y = _nrsqrt(r)
                r15 = r * r * y
                u15 = r15 * ws[sl] * wd[sl]
                damp = 1.0 - jnp.exp(-DAMP * u15)
                y2 = y * y
                rinv3 = y2 * y2 * y2
                coef[sl] = -(qd[sl] * damp) * rinv3
                return c2

            lax.fori_loop(0, C // 16, cbody, 0)

            def ebody(j, c2):
                sl = pl.ds(j * 16, 16)
                p = j * 16 + ii
                e = lax.div(p, 3)
                comp = p - e * 3
                g = plsc.load_gather(src_v, [e])
                idx3[sl] = g * 3 + comp
                val3[sl] = plsc.load_gather(coef, [e]) * vec_v[sl]
                return c2

            lax.fori_loop(0, (3 * C) // 16, ebody, 0)

            pltpu.sync_copy(val3, field_sp.at[idx3], add=True)
            return carry

        lax.fori_loop(0, NCHUNK, chunk_body, 0)

        plsc.subcore_barrier()

        @pl.when(sid == 0)
        def _():
            pltpu.sync_copy(field_sp, out_hbm.at[cid])

    return k(src, dst, dist, vecf, charges, pim, zeros)


def kernel(species, edge_src, edge_dst, distances, vec, charges, polarisability):
    del species
    src = edge_src.astype(jnp.int32)
    dst = edge_dst.astype(jnp.int32)
    dist = distances.astype(jnp.float32)
    vecf = vec.astype(jnp.float32).reshape(-1)

    pol_p = jnp.pad(polarisability.astype(jnp.float32), (0, N_PAD - N),
                    constant_values=1.0).reshape(N_PAD // 128 // 8, 8, 128)
    pim = pl.pallas_call(
        _prep_body,
        out_shape=jax.ShapeDtypeStruct(pol_p.shape, jnp.float32),
    )(pol_p).reshape(-1)[:N]

    zeros = jnp.zeros((3 * N,), jnp.float32)
    partials = _sc_field(src, dst, dist, vecf, charges.astype(jnp.float32),
                         pim, zeros)

    part_p = jnp.pad(partials, ((0, 0), (0, F_PAD - 3 * N)))
    part_p = part_p.reshape(2, F_PAD // 128 // 8, 8, 128)
    out = pl.pallas_call(
        _combine_body,
        out_shape=jax.ShapeDtypeStruct(part_p.shape[1:], jnp.float32),
    )(part_p)
    return out.reshape(-1)[:3 * N]


# speedup vs baseline: 177.5896x; 8.6021x over previous
import functools

import jax
import jax.numpy as jnp
from jax import lax
from jax.experimental import pallas as pl
from jax.experimental.pallas import tpu as pltpu
from jax.experimental.pallas import tpu_sc as plsc

N = 100000
E = 6400000
DAMP = 0.7

NC = 2
NS = 16
NW = NC * NS
EPW = E // NW
C = 2000
NCHUNK = EPW // C

N_PAD = 102400
F_PAD = 300032


def _prep_body(pol_ref, out_ref):
    x = pol_ref[...]
    out_ref[...] = lax.rsqrt(lax.sqrt(x))


def _combine_body(p_ref, out_ref):
    out_ref[...] = p_ref[0] + p_ref[1]


def _nrsqrt(x):
    i = lax.bitcast_convert_type(x, jnp.int32)
    i = jnp.int32(0x5F3759DF) - lax.shift_right_arithmetic(i, 1)
    y = lax.bitcast_convert_type(i, jnp.float32)
    for _ in range(3):
        y = y * (1.5 - 0.5 * x * y * y)
    return y


def _sc_field(src, dst, dist, vx, vy, vz, charges, pim, zeros):
    mesh = plsc.VectorSubcoreMesh(core_axis_name="c", subcore_axis_name="s")

    @functools.partial(
        pl.kernel,
        mesh=mesh,
        compiler_params=pltpu.CompilerParams(needs_layout_passes=False),
        out_type=jax.ShapeDtypeStruct((NC, 3 * N), jnp.float32),
        scratch_types=[
            pltpu.VMEM((C,), jnp.int32),
            pltpu.VMEM((C,), jnp.int32),
            pltpu.VMEM((C,), jnp.float32),
            pltpu.VMEM((C,), jnp.float32),
            pltpu.VMEM((C,), jnp.float32),
            pltpu.VMEM((C,), jnp.float32),
            pltpu.VMEM((C,), jnp.float32),
            pltpu.VMEM((C,), jnp.float32),
            pltpu.VMEM((C,), jnp.float32),
            pltpu.VMEM((C,), jnp.float32),
            pltpu.VMEM((C,), jnp.float32),
            pltpu.VMEM((C,), jnp.float32),
            pltpu.VMEM((C,), jnp.int32),
            pltpu.VMEM((C,), jnp.int32),
            pltpu.VMEM((C,), jnp.int32),
            pltpu.VMEM_SHARED((N,), jnp.float32),
            pltpu.VMEM_SHARED((N,), jnp.float32),
            pltpu.VMEM_SHARED((3 * N,), jnp.float32),
            pltpu.SemaphoreType.DMA,
        ],
    )
    def k(src_hbm, dst_hbm, dist_hbm, vx_hbm, vy_hbm, vz_hbm, charges_hbm,
          pim_hbm, zeros_hbm, out_hbm, src_v, dst_v, dist_v, vx_v, vy_v, vz_v,
          qd, ws, wd, ox, oy, oz, ix, iy, iz, charges_sp, pim_sp, field_sp,
          sem):
        cid = lax.axis_index("c")
        sid = lax.axis_index("s")
        wid = sid * NC + cid

        @pl.when(sid == 0)
        def _():
            pltpu.sync_copy(charges_hbm, charges_sp)
            pltpu.sync_copy(pim_hbm, pim_sp)
            pltpu.sync_copy(zeros_hbm, field_sp)

        plsc.subcore_barrier()

        def chunk_body(i, carry):
            base = wid * EPW + i * C
            sl_in = pl.ds(base, C)
            pltpu.sync_copy(src_hbm.at[sl_in], src_v)
            pltpu.sync_copy(dst_hbm.at[sl_in], dst_v)
            pltpu.sync_copy(dist_hbm.at[sl_in], dist_v)
            pltpu.sync_copy(vx_hbm.at[sl_in], vx_v)
            pltpu.sync_copy(vy_hbm.at[sl_in], vy_v)
            pltpu.sync_copy(vz_hbm.at[sl_in], vz_v)
            pltpu.async_copy(charges_sp.at[dst_v], qd, sem).wait()
            pltpu.async_copy(pim_sp.at[src_v], ws, sem).wait()
            pltpu.async_copy(pim_sp.at[dst_v], wd, sem).wait()

            def cbody(j, c2):
                sl = pl.ds(j * 16, 16)
                r = dist_v[sl]
                y = _nrsqrt(r)
                r15 = r * r * y
                u15 = r15 * ws[sl] * wd[sl]
                damp = 1.0 - jnp.exp(-DAMP * u15)
                y2 = y * y
                rinv3 = y2 * y2 * y2
                cf = -(qd[sl] * damp) * rinv3
                s3 = src_v[sl] * 3
                ix[sl] = s3
                iy[sl] = s3 + 1
                iz[sl] = s3 + 2
                ox[sl] = cf * vx_v[sl]
                oy[sl] = cf * vy_v[sl]
                oz[sl] = cf * vz_v[sl]
                return c2

            lax.fori_loop(0, C // 16, cbody, 0)

            pltpu.sync_copy(ox, field_sp.at[ix], add=True)
            pltpu.sync_copy(oy, field_sp.at[iy], add=True)
            pltpu.sync_copy(oz, field_sp.at[iz], add=True)
            return carry

        lax.fori_loop(0, NCHUNK, chunk_body, 0)

        plsc.subcore_barrier()

        @pl.when(sid == 0)
        def _():
            pltpu.sync_copy(field_sp, out_hbm.at[cid])

    return k(src, dst, dist, vx, vy, vz, charges, pim, zeros)


def kernel(species, edge_src, edge_dst, distances, vec, charges, polarisability):
    del species
    src = edge_src.astype(jnp.int32)
    dst = edge_dst.astype(jnp.int32)
    dist = distances.astype(jnp.float32)
    v32 = vec.astype(jnp.float32)
    vx, vy, vz = v32[:, 0], v32[:, 1], v32[:, 2]

    pol_p = jnp.pad(polarisability.astype(jnp.float32), (0, N_PAD - N),
                    constant_values=1.0).reshape(N_PAD // 128 // 8, 8, 128)
    pim = pl.pallas_call(
        _prep_body,
        out_shape=jax.ShapeDtypeStruct(pol_p.shape, jnp.float32),
    )(pol_p).reshape(-1)[:N]

    zeros = jnp.zeros((3 * N,), jnp.float32)
    partials = _sc_field(src, dst, dist, vx, vy, vz,
                         charges.astype(jnp.float32), pim, zeros)

    part_p = jnp.pad(partials, ((0, 0), (0, F_PAD - 3 * N)))
    part_p = part_p.reshape(2, F_PAD // 128 // 8, 8, 128)
    out = pl.pallas_call(
        _combine_body,
        out_shape=jax.ShapeDtypeStruct(part_p.shape[1:], jnp.float32),
    )(part_p)
    return out.reshape(-1)[:3 * N]


# speedup vs baseline: 200.8913x; 1.1312x over previous
import functools

import jax
import jax.numpy as jnp
from jax import lax
from jax.experimental import pallas as pl
from jax.experimental.pallas import tpu as pltpu
from jax.experimental.pallas import tpu_sc as plsc

N = 100000
E = 6400000
DAMP = 0.7

NC = 2
NS = 16
NW = NC * NS
EPW = E // NW
C = 2000
NCHUNK = EPW // C

N_PAD = 102400
F_PAD = 300032


def _prep_body(pol_ref, out_ref):
    x = pol_ref[...]
    out_ref[...] = lax.rsqrt(lax.sqrt(x))


def _combine_body(p_ref, out_ref):
    out_ref[...] = p_ref[0] + p_ref[1]


def _nrsqrt(x):
    i = lax.bitcast_convert_type(x, jnp.int32)
    i = jnp.int32(0x5F3759DF) - lax.shift_right_arithmetic(i, 1)
    y = lax.bitcast_convert_type(i, jnp.float32)
    for _ in range(3):
        y = y * (1.5 - 0.5 * x * y * y)
    return y


def _sc_field(src, dst, dist, vx, vy, vz, charges, pim, zeros):
    mesh = plsc.VectorSubcoreMesh(core_axis_name="c", subcore_axis_name="s")

    @functools.partial(
        pl.kernel,
        mesh=mesh,
        compiler_params=pltpu.CompilerParams(needs_layout_passes=False),
        out_type=jax.ShapeDtypeStruct((NC, 3 * N), jnp.float32),
        scratch_types=[
            [[pltpu.VMEM((C,), jnp.int32),
              pltpu.VMEM((C,), jnp.int32),
              pltpu.VMEM((C,), jnp.float32),
              pltpu.VMEM((C,), jnp.float32),
              pltpu.VMEM((C,), jnp.float32),
              pltpu.VMEM((C,), jnp.float32)]
             for _ in range(2)],
            pltpu.VMEM((C,), jnp.float32),
            pltpu.VMEM((C,), jnp.float32),
            pltpu.VMEM((C,), jnp.float32),
            pltpu.VMEM((C,), jnp.float32),
            pltpu.VMEM((C,), jnp.float32),
            pltpu.VMEM((C,), jnp.float32),
            pltpu.VMEM((C,), jnp.int32),
            pltpu.VMEM((C,), jnp.int32),
            pltpu.VMEM((C,), jnp.int32),
            pltpu.VMEM_SHARED((N,), jnp.float32),
            pltpu.VMEM_SHARED((N,), jnp.float32),
            pltpu.VMEM_SHARED((3 * N,), jnp.float32),
            [pltpu.SemaphoreType.DMA for _ in range(2)],
            pltpu.SemaphoreType.DMA,
            pltpu.SemaphoreType.DMA,
        ],
    )
    def k(src_hbm, dst_hbm, dist_hbm, vx_hbm, vy_hbm, vz_hbm, charges_hbm,
          pim_hbm, zeros_hbm, out_hbm, lin_bufs, qd, ws, wd, ox, oy, oz,
          ix, iy, iz, charges_sp, pim_sp, field_sp, lsems, gsem, ssem):
        cid = lax.axis_index("c")
        sid = lax.axis_index("s")
        wid = sid * NC + cid
        hbm_ins = [src_hbm, dst_hbm, dist_hbm, vx_hbm, vy_hbm, vz_hbm]

        @pl.when(sid == 0)
        def _():
            pltpu.sync_copy(charges_hbm, charges_sp)
            pltpu.sync_copy(pim_hbm, pim_sp)
            pltpu.sync_copy(zeros_hbm, field_sp)

        plsc.subcore_barrier()

        def issue_linear(ch, p):
            base = wid * EPW + ch * C
            sl_in = pl.ds(base, C)
            for h, v in zip(hbm_ins, lin_bufs[p]):
                pltpu.async_copy(h.at[sl_in], v, lsems[p])

        def drain_linear(p):
            for h, v in zip(hbm_ins, lin_bufs[p]):
                pltpu.make_async_copy(h.at[pl.ds(0, C)], v, lsems[p]).wait()

        def process(ch, p):
            drain_linear(p)
            src_v, dst_v, dist_v, vx_v, vy_v, vz_v = lin_bufs[p]
            g1 = pltpu.async_copy(charges_sp.at[dst_v], qd, gsem)
            g2 = pltpu.async_copy(pim_sp.at[src_v], ws, gsem)
            g3 = pltpu.async_copy(pim_sp.at[dst_v], wd, gsem)
            issue_linear(jnp.minimum(ch + 1, NCHUNK - 1), 1 - p)
            g1.wait()
            g2.wait()
            g3.wait()

            def cbody(j, c2):
                sl = pl.ds(j * 16, 16)
                r = dist_v[sl]
                y = _nrsqrt(r)
                r15 = r * r * y
                u15 = r15 * ws[sl] * wd[sl]
                damp = 1.0 - jnp.exp(-DAMP * u15)
                y2 = y * y
                rinv3 = y2 * y2 * y2
                cf = -(qd[sl] * damp) * rinv3
                s3 = src_v[sl] * 3
                ix[sl] = s3
                iy[sl] = s3 + 1
                iz[sl] = s3 + 2
                ox[sl] = cf * vx_v[sl]
                oy[sl] = cf * vy_v[sl]
                oz[sl] = cf * vz_v[sl]
                return c2

            lax.fori_loop(0, C // 16, cbody, 0)

            s1 = pltpu.async_copy(ox, field_sp.at[ix], ssem, add=True)
            s2 = pltpu.async_copy(oy, field_sp.at[iy], ssem, add=True)
            s3 = pltpu.async_copy(oz, field_sp.at[iz], ssem, add=True)
            s1.wait()
            s2.wait()
            s3.wait()

        issue_linear(0, 0)

        def body2(m, carry):
            process(2 * m, 0)
            process(2 * m + 1, 1)
            return carry

        lax.fori_loop(0, NCHUNK // 2, body2, 0)
        drain_linear(0)

        plsc.subcore_barrier()

        @pl.when(sid == 0)
        def _():
            pltpu.sync_copy(field_sp, out_hbm.at[cid])

    return k(src, dst, dist, vx, vy, vz, charges, pim, zeros)


def kernel(species, edge_src, edge_dst, distances, vec, charges, polarisability):
    del species
    src = edge_src.astype(jnp.int32)
    dst = edge_dst.astype(jnp.int32)
    dist = distances.astype(jnp.float32)
    v32 = vec.astype(jnp.float32)
    vx, vy, vz = v32[:, 0], v32[:, 1], v32[:, 2]

    pol_p = jnp.pad(polarisability.astype(jnp.float32), (0, N_PAD - N),
                    constant_values=1.0).reshape(N_PAD // 128 // 8, 8, 128)
    pim = pl.pallas_call(
        _prep_body,
        out_shape=jax.ShapeDtypeStruct(pol_p.shape, jnp.float32),
    )(pol_p).reshape(-1)[:N]

    zeros = jnp.zeros((3 * N,), jnp.float32)
    partials = _sc_field(src, dst, dist, vx, vy, vz,
                         charges.astype(jnp.float32), pim, zeros)

    part_p = jnp.pad(partials, ((0, 0), (0, F_PAD - 3 * N)))
    part_p = part_p.reshape(2, F_PAD // 128 // 8, 8, 128)
    out = pl.pallas_call(
        _combine_body,
        out_shape=jax.ShapeDtypeStruct(part_p.shape[1:], jnp.float32),
    )(part_p)
    return out.reshape(-1)[:3 * N]


# speedup vs baseline: 206.1216x; 1.0260x over previous
import functools

import jax
import jax.numpy as jnp
from jax import lax
from jax.experimental import pallas as pl
from jax.experimental.pallas import tpu as pltpu
from jax.experimental.pallas import tpu_sc as plsc

N = 100000
E = 6400000
DAMP = 0.7

NC = 2
NS = 16
NW = NC * NS
EPW = E // NW
C = 4000
NCHUNK = EPW // C

N_PAD = 102400
F_PAD = 300032


def _prep_body(pol_ref, out_ref):
    x = pol_ref[...]
    out_ref[...] = lax.rsqrt(lax.sqrt(x))


def _combine_body(p_ref, out_ref):
    out_ref[...] = p_ref[0] + p_ref[1]


def _nrsqrt(x):
    i = lax.bitcast_convert_type(x, jnp.int32)
    i = jnp.int32(0x5F3759DF) - lax.shift_right_arithmetic(i, 1)
    y = lax.bitcast_convert_type(i, jnp.float32)
    for _ in range(3):
        y = y * (1.5 - 0.5 * x * y * y)
    return y


def _sc_field(src, dst, dist, vx, vy, vz, charges, pim, zeros):
    mesh = plsc.VectorSubcoreMesh(core_axis_name="c", subcore_axis_name="s")

    @functools.partial(
        pl.kernel,
        mesh=mesh,
        compiler_params=pltpu.CompilerParams(needs_layout_passes=False),
        out_type=jax.ShapeDtypeStruct((NC, 3 * N), jnp.float32),
        scratch_types=[
            [[pltpu.VMEM((C,), jnp.int32),
              pltpu.VMEM((C,), jnp.int32),
              pltpu.VMEM((C,), jnp.float32),
              pltpu.VMEM((C,), jnp.float32),
              pltpu.VMEM((C,), jnp.float32),
              pltpu.VMEM((C,), jnp.float32)]
             for _ in range(2)],
            pltpu.VMEM((C,), jnp.float32),
            pltpu.VMEM((C,), jnp.float32),
            pltpu.VMEM((C,), jnp.float32),
            pltpu.VMEM((C,), jnp.float32),
            pltpu.VMEM((C,), jnp.float32),
            pltpu.VMEM((C,), jnp.float32),
            pltpu.VMEM((C,), jnp.int32),
            pltpu.VMEM((C,), jnp.int32),
            pltpu.VMEM((C,), jnp.int32),
            pltpu.VMEM_SHARED((N,), jnp.float32),
            pltpu.VMEM_SHARED((N,), jnp.float32),
            pltpu.VMEM_SHARED((3 * N,), jnp.float32),
            [pltpu.SemaphoreType.DMA for _ in range(2)],
            pltpu.SemaphoreType.DMA,
            pltpu.SemaphoreType.DMA,
        ],
    )
    def k(src_hbm, dst_hbm, dist_hbm, vx_hbm, vy_hbm, vz_hbm, charges_hbm,
          pim_hbm, zeros_hbm, out_hbm, lin_bufs, qd, ws, wd, ox, oy, oz,
          ix, iy, iz, charges_sp, pim_sp, field_sp, lsems, gsem, ssem):
        cid = lax.axis_index("c")
        sid = lax.axis_index("s")
        wid = sid * NC + cid
        hbm_ins = [src_hbm, dst_hbm, dist_hbm, vx_hbm, vy_hbm, vz_hbm]

        @pl.when(sid == 0)
        def _():
            pltpu.sync_copy(charges_hbm, charges_sp)
            pltpu.sync_copy(pim_hbm, pim_sp)
            pltpu.sync_copy(zeros_hbm, field_sp)

        plsc.subcore_barrier()

        def issue_linear(ch, p):
            base = wid * EPW + ch * C
            sl_in = pl.ds(base, C)
            for h, v in zip(hbm_ins, lin_bufs[p]):
                pltpu.async_copy(h.at[sl_in], v, lsems[p])

        def drain_linear(p):
            for h, v in zip(hbm_ins, lin_bufs[p]):
                pltpu.make_async_copy(h.at[pl.ds(0, C)], v, lsems[p]).wait()

        def process(ch, p):
            drain_linear(p)
            src_v, dst_v, dist_v, vx_v, vy_v, vz_v = lin_bufs[p]
            g1 = pltpu.async_copy(charges_sp.at[dst_v], qd, gsem)
            g2 = pltpu.async_copy(pim_sp.at[src_v], ws, gsem)
            g3 = pltpu.async_copy(pim_sp.at[dst_v], wd, gsem)
            issue_linear(jnp.minimum(ch + 1, NCHUNK - 1), 1 - p)
            g1.wait()
            g2.wait()
            g3.wait()

            def cbody(j, c2):
                sl = pl.ds(j * 16, 16)
                r = dist_v[sl]
                y = _nrsqrt(r)
                r15 = r * r * y
                u15 = r15 * ws[sl] * wd[sl]
                damp = 1.0 - jnp.exp(-DAMP * u15)
                y2 = y * y
                rinv3 = y2 * y2 * y2
                cf = -(qd[sl] * damp) * rinv3
                s3 = src_v[sl] * 3
                ix[sl] = s3
                iy[sl] = s3 + 1
                iz[sl] = s3 + 2
                ox[sl] = cf * vx_v[sl]
                oy[sl] = cf * vy_v[sl]
                oz[sl] = cf * vz_v[sl]
                return c2

            lax.fori_loop(0, C // 16, cbody, 0)

            s1 = pltpu.async_copy(ox, field_sp.at[ix], ssem, add=True)
            s2 = pltpu.async_copy(oy, field_sp.at[iy], ssem, add=True)
            s3 = pltpu.async_copy(oz, field_sp.at[iz], ssem, add=True)
            s1.wait()
            s2.wait()
            s3.wait()

        issue_linear(0, 0)

        def body2(m, carry):
            process(2 * m, 0)
            process(2 * m + 1, 1)
            return carry

        lax.fori_loop(0, NCHUNK // 2, body2, 0)
        drain_linear(0)

        plsc.subcore_barrier()

        @pl.when(sid == 0)
        def _():
            pltpu.sync_copy(field_sp, out_hbm.at[cid])

    return k(src, dst, dist, vx, vy, vz, charges, pim, zeros)


def kernel(species, edge_src, edge_dst, distances, vec, charges, polarisability):
    del species
    src = edge_src.astype(jnp.int32)
    dst = edge_dst.astype(jnp.int32)
    dist = distances.astype(jnp.float32)
    v32 = vec.astype(jnp.float32)
    vx, vy, vz = v32[:, 0], v32[:, 1], v32[:, 2]

    pol_p = jnp.pad(polarisability.astype(jnp.float32), (0, N_PAD - N),
                    constant_values=1.0).reshape(N_PAD // 128 // 8, 8, 128)
    pim = pl.pallas_call(
        _prep_body,
        out_shape=jax.ShapeDtypeStruct(pol_p.shape, jnp.float32),
    )(pol_p).reshape(-1)[:N]

    zeros = jnp.zeros((3 * N,), jnp.float32)
    partials = _sc_field(src, dst, dist, vx, vy, vz,
                         charges.astype(jnp.float32), pim, zeros)

    part_p = jnp.pad(partials, ((0, 0), (0, F_PAD - 3 * N)))
    part_p = part_p.reshape(2, F_PAD // 128 // 8, 8, 128)
    out = pl.pallas_call(
        _combine_body,
        out_shape=jax.ShapeDtypeStruct(part_p.shape[1:], jnp.float32),
    )(part_p)
    return out.reshape(-1)[:3 * N]
